# baseline (speedup 1.0000x reference)
import functools

import jax
import jax.numpy as jnp
from jax import lax
from jax.experimental import pallas as pl
from jax.experimental.pallas import tpu as pltpu
from jax.experimental.pallas import tpu_sc as plsc

_B = 16; _E = 16384; _D = 256; _H = 8; _DH = 32; _L = 128; _DEPTH = 2
_NN = 4096; _NT = 2048; _NV = 64; _BD = 2; _DFF = 1024
_CHUNK = 512
_NCHUNK = _E // _CHUNK
_SC_CHUNK = 128
_SCALE = 1.0 / (float(_DH) ** 0.5)
_NWC = 9
_NWT = 34


def _sc_gather(neuron_ids, time_bins, values, ntab, ttab, vtab):
    info = plsc.get_sparse_core_info()
    nw = info.num_cores * info.num_subcores
    epw = _E // nw
    nch = epw // _SC_CHUNK
    mesh = plsc.VectorSubcoreMesh(core_axis_name="c", subcore_axis_name="s")

    def body(nid_h, tid_h, vid_h, nt_h, tt_h, vt_h, on_h, ot_h, ov_h,
             i0, i1, i2, r0, r1, r2, sem):
        wid = lax.axis_index("s") * info.num_cores + lax.axis_index("c")
        for c in range(nch):
            base = wid * epw + c * _SC_CHUNK
            pltpu.sync_copy(nid_h.at[pl.ds(base, _SC_CHUNK)], i0)
            pltpu.sync_copy(tid_h.at[pl.ds(base, _SC_CHUNK)], i1)
            pltpu.sync_copy(vid_h.at[pl.ds(base, _SC_CHUNK)], i2)
            c0 = pltpu.async_copy(nt_h.at[i0], r0, sem)
            c1 = pltpu.async_copy(tt_h.at[i1], r1, sem)
            c2 = pltpu.async_copy(vt_h.at[i2], r2, sem)
            c0.wait(); c1.wait(); c2.wait()
            pltpu.sync_copy(r0, on_h.at[pl.ds(base, _SC_CHUNK)])
            pltpu.sync_copy(r1, ot_h.at[pl.ds(base, _SC_CHUNK)])
            pltpu.sync_copy(r2, ov_h.at[pl.ds(base, _SC_CHUNK)])

    f = pl.kernel(
        body,
        out_type=[jax.ShapeDtypeStruct((_E, _D), jnp.float32) for _ in range(3)],
        mesh=mesh,
        scratch_types=[pltpu.VMEM((_SC_CHUNK,), jnp.int32) for _ in range(3)]
        + [pltpu.VMEM((_SC_CHUNK, _D), jnp.float32) for _ in range(3)]
        + [pltpu.SemaphoreType.DMA],
    )
    return f(neuron_ids, time_bins, values, ntab, ttab, vtab)


def _ln(x, g, b):
    mu = jnp.mean(x, axis=-1, keepdims=True)
    var = jnp.mean((x - mu) ** 2, axis=-1, keepdims=True)
    return (x - mu) * lax.rsqrt(var + 1e-5) * g + b


_PREC = lax.Precision.HIGHEST


def _mm(a, b):
    return jnp.dot(a, b, precision=_PREC, preferred_element_type=jnp.float32)


def _mm_t(a, b):
    return lax.dot_general(a, b, (((1,), (1,)), ((), ())),
                           precision=_PREC, preferred_element_type=jnp.float32)


def _block_diag(q, rows):
    qv = jnp.concatenate([q] * _H, axis=0)
    row_blk = lax.broadcasted_iota(jnp.int32, (_H * rows, 1), 0) // rows
    col_blk = lax.broadcasted_iota(jnp.int32, (1, _D), 1) // _DH
    return jnp.where(row_blk == col_blk, qv, 0.0)


def _unblock(o, rows):
    return jnp.concatenate(
        [o[h * rows:(h + 1) * rows, h * _DH:(h + 1) * _DH] for h in range(_H)],
        axis=1)


def _cross_body(bi_ref, xn_ref, xt_ref, xv_ref, *rest):
    w = rest[:_NWC]
    out_ref = rest[_NWC]
    qbig_ref, acc_ref, ls_ref = rest[_NWC + 1:]
    j = pl.program_id(0)
    (latents_r, tokg, tokb, lnqg, lnqb, lnkg, lnkb, cwq, cwkv) = w
    hl = _H * _L

    @pl.when(j == 0)
    def _init():
        qq = _mm(_ln(latents_r[...], lnqg[...], lnqb[...]), cwq[...]) * _SCALE
        qbig_ref[...] = _block_diag(qq, _L)
        acc_ref[...] = jnp.zeros_like(acc_ref)
        ls_ref[...] = jnp.zeros_like(ls_ref)

    x = xn_ref[...] + xt_ref[...] + xv_ref[...]
    x = _ln(x, tokg[...], tokb[...])
    y = _ln(x, lnkg[...], lnkb[...])
    kv = _mm(y, cwkv[...])
    kk = kv[:, :_D]
    vv = kv[:, _D:]
    brow = bi_ref[...].reshape(1, _CHUNK)
    p = jnp.exp(_mm_t(qbig_ref[...], kk))
    bfirst = jnp.min(brow)
    blast = jnp.max(brow)

    def per_b(bb, carry):
        pb = jnp.where(brow == bb, p, 0.0)
        base = bb * hl
        acc_ref[pl.ds(base, hl), :] = acc_ref[pl.ds(base, hl), :] + _mm(pb, vv)
        ls_ref[pl.ds(base, hl), :] = ls_ref[pl.ds(base, hl), :] + \
            jnp.sum(pb, axis=1, keepdims=True)
        return carry

    lax.fori_loop(bfirst, blast + 1, per_b, 0)

    @pl.when(j == _NCHUNK - 1)
    def _fin():
        for bb in range(_B):
            a = acc_ref[bb * hl:(bb + 1) * hl, :]
            l = ls_ref[bb * hl:(bb + 1) * hl, :]
            a = jnp.where(l > 0, a / jnp.maximum(l, 1e-30), 0.0)
            out_ref[bb, :, :] = _unblock(a, _L)


def _tc_cross(bi3d, pn, pt, pv, weights):
    in_specs = (
        [pl.BlockSpec((1, 1, _CHUNK), lambda j: (j, 0, 0))]
        + [pl.BlockSpec((_CHUNK, _D), lambda j: (j, 0))] * 3
        + [pl.BlockSpec(wa.shape, functools.partial(lambda j, n: (0,) * n, n=wa.ndim))
           for wa in weights]
    )
    return pl.pallas_call(
        _cross_body,
        grid=(_NCHUNK,),
        in_specs=in_specs,
        out_specs=pl.BlockSpec((_B, _L, _D), lambda j: (0, 0, 0)),
        out_shape=jax.ShapeDtypeStruct((_B, _L, _D), jnp.float32),
        scratch_shapes=[pltpu.VMEM((_H * _L, _D), jnp.float32),
                        pltpu.VMEM((_B * _H * _L, _D), jnp.float32),
                        pltpu.VMEM((_B * _H * _L, 1), jnp.float32)],
    )(bi3d, pn, pt, pv, *weights)


def _softmax_rows(s):
    p = jnp.exp(s)
    return p / jnp.sum(p, axis=-1, keepdims=True)


def _tail_body(attn_ref, *rest):
    w = rest[:_NWT]
    out_ref = rest[_NWT]
    (latents_r, cwo, cln2g, cln2b, cw1, cb1, cw2, cb2) = w[:8]

    attn = attn_ref[...].reshape(_B * _L, _D)
    latb = jnp.concatenate([latents_r[...]] * _B, axis=0)
    lat = latb + _mm(attn, cwo[...])
    hh = _ln(lat, cln2g[...], cln2b[...])
    lat = lat + _mm(jax.nn.gelu(_mm(hh, cw1[...]) + cb1[...]), cw2[...]) + cb2[...]

    for li in range(_DEPTH):
        (l1g, l1b, wqkv, lwo, l2g, l2b, lw1, lb1, lw2, lb2) = \
            w[8 + 10 * li:18 + 10 * li]
        hh = _ln(lat, l1g[...], l1b[...])
        qkv = _mm(hh, wqkv[...])
        merged = []
        for bb in range(_B):
            qb_ = qkv[bb * _L:(bb + 1) * _L, :_D] * _SCALE
            kb_ = qkv[bb * _L:(bb + 1) * _L, _D:2 * _D]
            vb_ = qkv[bb * _L:(bb + 1) * _L, 2 * _D:]
            p = _softmax_rows(_mm_t(_block_diag(qb_, _L), kb_))
            merged.append(_unblock(_mm(p, vb_), _L))
        lat = lat + _mm(jnp.concatenate(merged, axis=0), lwo[...])
        hh = _ln(lat, l2g[...], l2b[...])
        lat = lat + _mm(jax.nn.gelu(_mm(hh, lw1[...]) + lb1[...]), lw2[...]) + lb2[...]

    (bquery, bwq, bwkv, bwoT, blng, blnb) = w[28:34]
    nl2 = _ln(lat, blng[...], blnb[...])
    kvb = _mm(nl2, bwkv[...])
    qb = _mm(bquery[...], bwq[...]) * _SCALE
    qbig = _block_diag(qb, _BD)
    bwo_row = bwoT[...]
    rows = []
    for bb in range(_B):
        kb_ = kvb[bb * _L:(bb + 1) * _L, :_D]
        vb_ = kvb[bb * _L:(bb + 1) * _L, _D:]
        p = _softmax_rows(_mm_t(qbig, kb_))
        o = _mm(p, vb_)
        attnb = jnp.concatenate(
            [o[h * _BD:(h + 1) * _BD, h * _DH:(h + 1) * _DH] for h in range(_H)],
            axis=1)
        prod = attnb * bwo_row
        vals = [jnp.sum(prod[d:d + 1, :], axis=1, keepdims=True)
                for d in range(_BD)]
        rows.append(jnp.concatenate(vals, axis=1))
    out_ref[...] = jnp.concatenate(rows, axis=0)


def _tc_tail(attn_all, weights):
    return pl.pallas_call(
        _tail_body,
        out_shape=jax.ShapeDtypeStruct((_B, _BD), jnp.float32),
    )(attn_all, *weights)


def kernel(params, neuron_ids, time_bins, values, batch_indices):
    p = params
    pn, pt, pv = _sc_gather(neuron_ids, time_bins, values,
                            p['neuron_emb'], p['time_emb'], p['value_emb'])
    r = lambda a: a.reshape(1, -1)
    c = p['cross']
    bh = p['beh']
    cross_w = [p['latents'], r(p['tok_ln_g']), r(p['tok_ln_b']),
               r(c['lnq_g']), r(c['lnq_b']), r(c['lnk_g']), r(c['lnk_b']),
               c['wq'], c['wkv']]
    tail_w = [p['latents'], c['wo'], r(c['ln2_g']), r(c['ln2_b']),
              c['w1'], r(c['b1']), c['w2'], r(c['b2'])]
    for lyr in p['layers']:
        tail_w += [r(lyr['ln1_g']), r(lyr['ln1_b']), lyr['wqkv'], lyr['wo'],
                   r(lyr['ln2_g']), r(lyr['ln2_b']), lyr['w1'], r(lyr['b1']),
                   lyr['w2'], r(lyr['b2'])]
    tail_w += [bh['query'], bh['wq'], bh['wkv'], bh['wo'].reshape(1, _D),
               r(bh['ln_g']), r(bh['ln_b'])]
    assert len(cross_w) == _NWC and len(tail_w) == _NWT
    bi3d = batch_indices.astype(jnp.int32).reshape(_NCHUNK, 1, _CHUNK)
    attn_all = _tc_cross(bi3d, pn, pt, pv, cross_w)
    return _tc_tail(attn_all, tail_w)


# speedup vs baseline: 1.8677x; 1.8677x over previous
import functools

import jax
import jax.numpy as jnp
from jax import lax
from jax.experimental import pallas as pl
from jax.experimental.pallas import tpu as pltpu
from jax.experimental.pallas import tpu_sc as plsc

_B = 16; _E = 16384; _D = 256; _H = 8; _DH = 32; _L = 128; _DEPTH = 2
_NN = 4096; _NT = 2048; _NV = 64; _BD = 2; _DFF = 1024
_CHUNK = 512
_NCHUNK = _E // _CHUNK
_SC_CHUNK = 128
_SCALE = 1.0 / (float(_DH) ** 0.5)
_NWC = 9
_NWT = 34


def _sc_gather(neuron_ids, time_bins, values, ntab, ttab, vtab):
    info = plsc.get_sparse_core_info()
    nw = info.num_cores * info.num_subcores
    epw = _E // nw
    nch = epw // _SC_CHUNK
    mesh = plsc.VectorSubcoreMesh(core_axis_name="c", subcore_axis_name="s")

    def body(nid_h, tid_h, vid_h, nt_h, tt_h, vt_h, on_h, ot_h, ov_h,
             i0, i1, i2, r0, r1, r2, sem):
        wid = lax.axis_index("s") * info.num_cores + lax.axis_index("c")
        for c in range(nch):
            base = wid * epw + c * _SC_CHUNK
            pltpu.sync_copy(nid_h.at[pl.ds(base, _SC_CHUNK)], i0)
            pltpu.sync_copy(tid_h.at[pl.ds(base, _SC_CHUNK)], i1)
            pltpu.sync_copy(vid_h.at[pl.ds(base, _SC_CHUNK)], i2)
            c0 = pltpu.async_copy(nt_h.at[i0], r0, sem)
            c1 = pltpu.async_copy(tt_h.at[i1], r1, sem)
            c2 = pltpu.async_copy(vt_h.at[i2], r2, sem)
            c0.wait(); c1.wait(); c2.wait()
            pltpu.sync_copy(r0, on_h.at[pl.ds(base, _SC_CHUNK)])
            pltpu.sync_copy(r1, ot_h.at[pl.ds(base, _SC_CHUNK)])
            pltpu.sync_copy(r2, ov_h.at[pl.ds(base, _SC_CHUNK)])

    f = pl.kernel(
        body,
        out_type=[jax.ShapeDtypeStruct((_E, _D), jnp.float32) for _ in range(3)],
        mesh=mesh,
        scratch_types=[pltpu.VMEM((_SC_CHUNK,), jnp.int32) for _ in range(3)]
        + [pltpu.VMEM((_SC_CHUNK, _D), jnp.float32) for _ in range(3)]
        + [pltpu.SemaphoreType.DMA],
    )
    return f(neuron_ids, time_bins, values, ntab, ttab, vtab)


def _ln(x, g, b):
    mu = jnp.mean(x, axis=-1, keepdims=True)
    var = jnp.mean((x - mu) ** 2, axis=-1, keepdims=True)
    return (x - mu) / jnp.sqrt(var + 1e-5) * g + b


def _mm(a, b):
    return jnp.dot(a, b, preferred_element_type=jnp.float32)


def _mm_t(a, b):
    return lax.dot_general(a, b, (((1,), (1,)), ((), ())),
                           preferred_element_type=jnp.float32)


_SQRTDH = float(jnp.sqrt(jnp.float32(_DH)))


def _block_diag(q, rows):
    qv = jnp.concatenate([q] * _H, axis=0)
    row_blk = lax.broadcasted_iota(jnp.int32, (_H * rows, 1), 0) // rows
    col_blk = lax.broadcasted_iota(jnp.int32, (1, _D), 1) // _DH
    return jnp.where(row_blk == col_blk, qv, 0.0)


def _unblock(o, rows):
    return jnp.concatenate(
        [o[h * rows:(h + 1) * rows, h * _DH:(h + 1) * _DH] for h in range(_H)],
        axis=1)


def _cross_body(bi_ref, xn_ref, xt_ref, xv_ref, *rest):
    w = rest[:_NWC]
    out_ref = rest[_NWC]
    qbig_ref, acc_ref, ls_ref = rest[_NWC + 1:]
    ph = pl.program_id(0)
    j = pl.program_id(1)
    (latents_r, tokg, tokb, lnqg, lnqb, lnkg, lnkb, cwq, cwkv) = w
    hl = _H * _L

    @pl.when((ph == 0) & (j == 0))
    def _init():
        qq = _mm(_ln(latents_r[...], lnqg[...], lnqb[...]), cwq[...])
        qbig_ref[...] = _block_diag(qq, _L)
        acc_ref[...] = jnp.zeros_like(acc_ref)
        ls_ref[...] = jnp.zeros_like(ls_ref)

    x = xn_ref[...] + xt_ref[...] + xv_ref[...]
    x = _ln(x, tokg[...], tokb[...])
    y = _ln(x, lnkg[...], lnkb[...])
    kv = _mm(y, cwkv[...])
    kk = kv[:, :_D]
    vv = kv[:, _D:]
    brow = bi_ref[...].reshape(1, _CHUNK)
    e = jnp.exp(_mm_t(qbig_ref[...], kk) / _SQRTDH)
    bfirst = jnp.min(brow)
    blast = jnp.max(brow)

    @pl.when(ph == 0)
    def _denoms():
        def per_b(bb, carry):
            eb = jnp.where(brow == bb, e, 0.0)
            base = bb * hl
            ls_ref[pl.ds(base, hl), :] = ls_ref[pl.ds(base, hl), :] + \
                jnp.sum(eb, axis=1, keepdims=True)
            return carry
        lax.fori_loop(bfirst, blast + 1, per_b, 0)

    @pl.when(ph == 1)
    def _attend():
        def per_b(bb, carry):
            base = bb * hl
            lb = ls_ref[pl.ds(base, hl), :]
            pb = jnp.where(brow == bb, e / lb, 0.0)
            acc_ref[pl.ds(base, hl), :] = acc_ref[pl.ds(base, hl), :] + \
                _mm(pb, vv)
            return carry
        lax.fori_loop(bfirst, blast + 1, per_b, 0)

    @pl.when((ph == 1) & (j == _NCHUNK - 1))
    def _fin():
        for bb in range(_B):
            out_ref[bb, :, :] = _unblock(acc_ref[bb * hl:(bb + 1) * hl, :], _L)


def _tc_cross(bi3d, pn, pt, pv, weights):
    in_specs = (
        [pl.BlockSpec((1, 1, _CHUNK), lambda ph, j: (j, 0, 0))]
        + [pl.BlockSpec((_CHUNK, _D), lambda ph, j: (j, 0))] * 3
        + [pl.BlockSpec(wa.shape, functools.partial(lambda ph, j, n: (0,) * n, n=wa.ndim))
           for wa in weights]
    )
    return pl.pallas_call(
        _cross_body,
        grid=(2, _NCHUNK),
        in_specs=in_specs,
        out_specs=pl.BlockSpec((_B, _L, _D), lambda ph, j: (0, 0, 0)),
        out_shape=jax.ShapeDtypeStruct((_B, _L, _D), jnp.float32),
        scratch_shapes=[pltpu.VMEM((_H * _L, _D), jnp.float32),
                        pltpu.VMEM((_B * _H * _L, _D), jnp.float32),
                        pltpu.VMEM((_B * _H * _L, 1), jnp.float32)],
    )(bi3d, pn, pt, pv, *weights)


def _softmax_rows(s):
    p = jnp.exp(s)
    return p / jnp.sum(p, axis=-1, keepdims=True)


def _tail_body(attn_ref, *rest):
    w = rest[:_NWT]
    out_ref = rest[_NWT]
    (latents_r, cwo, cln2g, cln2b, cw1, cb1, cw2, cb2) = w[:8]

    attn = attn_ref[...].reshape(_B * _L, _D)
    latb = jnp.concatenate([latents_r[...]] * _B, axis=0)
    lat = latb + _mm(attn, cwo[...])
    hh = _ln(lat, cln2g[...], cln2b[...])
    lat = lat + _mm(jax.nn.gelu(_mm(hh, cw1[...]) + cb1[...]), cw2[...]) + cb2[...]

    for li in range(_DEPTH):
        (l1g, l1b, wqkv, lwo, l2g, l2b, lw1, lb1, lw2, lb2) = \
            w[8 + 10 * li:18 + 10 * li]
        hh = _ln(lat, l1g[...], l1b[...])
        qkv = _mm(hh, wqkv[...])
        merged = []
        for bb in range(_B):
            qb_ = qkv[bb * _L:(bb + 1) * _L, :_D]
            kb_ = qkv[bb * _L:(bb + 1) * _L, _D:2 * _D]
            vb_ = qkv[bb * _L:(bb + 1) * _L, 2 * _D:]
            p = _softmax_rows(_mm_t(_block_diag(qb_, _L), kb_) / _SQRTDH)
            merged.append(_unblock(_mm(p, vb_), _L))
        lat = lat + _mm(jnp.concatenate(merged, axis=0), lwo[...])
        hh = _ln(lat, l2g[...], l2b[...])
        lat = lat + _mm(jax.nn.gelu(_mm(hh, lw1[...]) + lb1[...]), lw2[...]) + lb2[...]

    (bquery, bwq, bwkv, bwoT, blng, blnb) = w[28:34]
    nl2 = _ln(lat, blng[...], blnb[...])
    kvb = _mm(nl2, bwkv[...])
    qb = _mm(bquery[...], bwq[...])
    qbig = _block_diag(qb, _BD)
    bwo_row = bwoT[...].astype(jnp.bfloat16).astype(jnp.float32)
    rows = []
    for bb in range(_B):
        kb_ = kvb[bb * _L:(bb + 1) * _L, :_D]
        vb_ = kvb[bb * _L:(bb + 1) * _L, _D:]
        p = _softmax_rows(_mm_t(qbig, kb_) / _SQRTDH)
        o = _mm(p, vb_)
        attnb = jnp.concatenate(
            [o[h * _BD:(h + 1) * _BD, h * _DH:(h + 1) * _DH] for h in range(_H)],
            axis=1)
        prod = attnb.astype(jnp.bfloat16).astype(jnp.float32) * bwo_row
        vals = [jnp.sum(prod[d:d + 1, :], axis=1, keepdims=True)
                for d in range(_BD)]
        rows.append(jnp.concatenate(vals, axis=1))
    out_ref[...] = jnp.concatenate(rows, axis=0)


def _tc_tail(attn_all, weights):
    return pl.pallas_call(
        _tail_body,
        out_shape=jax.ShapeDtypeStruct((_B, _BD), jnp.float32),
    )(attn_all, *weights)


def kernel(params, neuron_ids, time_bins, values, batch_indices):
    p = params
    pn, pt, pv = _sc_gather(neuron_ids, time_bins, values,
                            p['neuron_emb'], p['time_emb'], p['value_emb'])
    r = lambda a: a.reshape(1, -1)
    c = p['cross']
    bh = p['beh']
    cross_w = [p['latents'], r(p['tok_ln_g']), r(p['tok_ln_b']),
               r(c['lnq_g']), r(c['lnq_b']), r(c['lnk_g']), r(c['lnk_b']),
               c['wq'], c['wkv']]
    tail_w = [p['latents'], c['wo'], r(c['ln2_g']), r(c['ln2_b']),
              c['w1'], r(c['b1']), c['w2'], r(c['b2'])]
    for lyr in p['layers']:
        tail_w += [r(lyr['ln1_g']), r(lyr['ln1_b']), lyr['wqkv'], lyr['wo'],
                   r(lyr['ln2_g']), r(lyr['ln2_b']), lyr['w1'], r(lyr['b1']),
                   lyr['w2'], r(lyr['b2'])]
    tail_w += [bh['query'], bh['wq'], bh['wkv'], bh['wo'].reshape(1, _D),
               r(bh['ln_g']), r(bh['ln_b'])]
    assert len(cross_w) == _NWC and len(tail_w) == _NWT
    bi3d = batch_indices.astype(jnp.int32).reshape(_NCHUNK, 1, _CHUNK)
    attn_all = _tc_cross(bi3d, pn, pt, pv, cross_w)
    return _tc_tail(attn_all, tail_w)


# speedup vs baseline: 1.8958x; 1.0151x over previous
import functools

import jax
import jax.numpy as jnp
from jax import lax
from jax.experimental import pallas as pl
from jax.experimental.pallas import tpu as pltpu
from jax.experimental.pallas import tpu_sc as plsc

_B = 16; _E = 16384; _D = 256; _H = 8; _DH = 32; _L = 128; _DEPTH = 2
_NN = 4096; _NT = 2048; _NV = 64; _BD = 2; _DFF = 1024
_CHUNK = 512
_NCHUNK = _E // _CHUNK
_SC_CHUNK = 128
_SCALE = 1.0 / (float(_DH) ** 0.5)
_NWC = 10
_NWT = 34


def _sc_gather(neuron_ids, time_bins, values, ntab, ttab, vtab):
    info = plsc.get_sparse_core_info()
    nw = info.num_cores * info.num_subcores
    epw = _E // nw
    nch = epw // _SC_CHUNK
    mesh = plsc.VectorSubcoreMesh(core_axis_name="c", subcore_axis_name="s")

    def body(nid_h, tid_h, vid_h, nt_h, tt_h, vt_h, out_h,
             i0, i1, i2, r0, r1, r2, sem):
        wid = lax.axis_index("s") * info.num_cores + lax.axis_index("c")
        for c in range(nch):
            base = wid * epw + c * _SC_CHUNK
            pltpu.sync_copy(nid_h.at[pl.ds(base, _SC_CHUNK)], i0)
            pltpu.sync_copy(tid_h.at[pl.ds(base, _SC_CHUNK)], i1)
            pltpu.sync_copy(vid_h.at[pl.ds(base, _SC_CHUNK)], i2)
            c0 = pltpu.async_copy(nt_h.at[i0], r0, sem)
            c1 = pltpu.async_copy(tt_h.at[i1], r1, sem)
            c2 = pltpu.async_copy(vt_h.at[i2], r2, sem)
            c0.wait(); c1.wait(); c2.wait()

            def row(rr, carry):
                for c16 in range(_D // 16):
                    sl = pl.ds(c16 * 16, 16)
                    r0[rr, sl] = (r0[rr, sl] + r1[rr, sl]) + r2[rr, sl]
                return carry
            lax.fori_loop(0, _SC_CHUNK, row, 0)
            pltpu.sync_copy(r0, out_h.at[pl.ds(base, _SC_CHUNK)])

    f = pl.kernel(
        body,
        out_type=jax.ShapeDtypeStruct((_E, _D), jnp.float32),
        mesh=mesh,
        scratch_types=[pltpu.VMEM((_SC_CHUNK,), jnp.int32) for _ in range(3)]
        + [pltpu.VMEM((_SC_CHUNK, _D), jnp.float32) for _ in range(3)]
        + [pltpu.SemaphoreType.DMA],
    )
    return f(neuron_ids, time_bins, values, ntab, ttab, vtab)


def _ln(x, g, b):
    mu = jnp.mean(x, axis=-1, keepdims=True)
    var = jnp.mean((x - mu) ** 2, axis=-1, keepdims=True)
    return (x - mu) / jnp.sqrt(var + 1e-5) * g + b


def _mm(a, b):
    return jnp.dot(a, b, preferred_element_type=jnp.float32)


def _mm_t(a, b):
    return lax.dot_general(a, b, (((1,), (1,)), ((), ())),
                           preferred_element_type=jnp.float32)


_SQRTDH = float(_DH) ** 0.5


def _block_diag(q, rows):
    qv = jnp.concatenate([q] * _H, axis=0)
    row_blk = lax.broadcasted_iota(jnp.int32, (_H * rows, 1), 0) // rows
    col_blk = lax.broadcasted_iota(jnp.int32, (1, _D), 1) // _DH
    return jnp.where(row_blk == col_blk, qv, 0.0)


def _unblock(o, rows):
    return jnp.concatenate(
        [o[h * rows:(h + 1) * rows, h * _DH:(h + 1) * _DH] for h in range(_H)],
        axis=1)


def _cross_body(bi_ref, x_ref, *rest):
    w = rest[:_NWC]
    out_ref = rest[_NWC]
    qbig_ref, acc_ref, ls_ref = rest[_NWC + 1:]
    ph = pl.program_id(0)
    j = pl.program_id(1)
    (latents_r, tokg, tokb, lnqg, lnqb, lnkg, lnkb, cwq, cwk, cwv) = w
    hl = _H * _L

    @pl.when((ph == 0) & (j == 0))
    def _init():
        qq = _mm(_ln(latents_r[...], lnqg[...], lnqb[...]), cwq[...])
        qbig_ref[...] = _block_diag(qq, _L)
        acc_ref[...] = jnp.zeros_like(acc_ref)
        ls_ref[...] = jnp.zeros_like(ls_ref)

    x = _ln(x_ref[...], tokg[...], tokb[...])
    y = _ln(x, lnkg[...], lnkb[...])
    kk = _mm(y, cwk[...])
    brow = bi_ref[...].reshape(1, _CHUNK)
    e = jnp.exp(_mm_t(qbig_ref[...], kk) / _SQRTDH)
    bfirst = jnp.min(brow)
    blast = jnp.max(brow)

    @pl.when(ph == 0)
    def _denoms():
        def per_b(bb, carry):
            eb = jnp.where(brow == bb, e, 0.0)
            base = bb * hl
            ls_ref[pl.ds(base, hl), :] = ls_ref[pl.ds(base, hl), :] + \
                jnp.sum(eb, axis=1, keepdims=True)
            return carry
        lax.fori_loop(bfirst, blast + 1, per_b, 0)

    @pl.when(ph == 1)
    def _attend():
        vv = _mm(y, cwv[...])

        def per_b(bb, carry):
            base = bb * hl
            lb = ls_ref[pl.ds(base, hl), :]
            pb = jnp.where(brow == bb, e / lb, 0.0)
            acc_ref[pl.ds(base, hl), :] = acc_ref[pl.ds(base, hl), :] + \
                _mm(pb, vv)
            return carry
        lax.fori_loop(bfirst, blast + 1, per_b, 0)

    @pl.when((ph == 1) & (j == _NCHUNK - 1))
    def _fin():
        for bb in range(_B):
            out_ref[bb, :, :] = _unblock(acc_ref[bb * hl:(bb + 1) * hl, :], _L)


def _tc_cross(bi3d, flat, weights):
    in_specs = (
        [pl.BlockSpec((1, 1, _CHUNK), lambda ph, j: (j, 0, 0)),
         pl.BlockSpec((_CHUNK, _D), lambda ph, j: (j, 0))]
        + [pl.BlockSpec(wa.shape, functools.partial(lambda ph, j, n: (0,) * n, n=wa.ndim))
           for wa in weights]
    )
    return pl.pallas_call(
        _cross_body,
        grid=(2, _NCHUNK),
        in_specs=in_specs,
        out_specs=pl.BlockSpec((_B, _L, _D), lambda ph, j: (0, 0, 0)),
        out_shape=jax.ShapeDtypeStruct((_B, _L, _D), jnp.float32),
        scratch_shapes=[pltpu.VMEM((_H * _L, _D), jnp.float32),
                        pltpu.VMEM((_B * _H * _L, _D), jnp.float32),
                        pltpu.VMEM((_B * _H * _L, 1), jnp.float32)],
    )(bi3d, flat, *weights)


def _softmax_rows(s):
    p = jnp.exp(s)
    return p / jnp.sum(p, axis=-1, keepdims=True)


def _tail_body(attn_ref, *rest):
    w = rest[:_NWT]
    out_ref = rest[_NWT]
    (latents_r, cwo, cln2g, cln2b, cw1, cb1, cw2, cb2) = w[:8]

    attn = attn_ref[...].reshape(_B * _L, _D)
    latb = jnp.concatenate([latents_r[...]] * _B, axis=0)
    lat = latb + _mm(attn, cwo[...])
    hh = _ln(lat, cln2g[...], cln2b[...])
    lat = lat + _mm(jax.nn.gelu(_mm(hh, cw1[...]) + cb1[...]), cw2[...]) + cb2[...]

    for li in range(_DEPTH):
        (l1g, l1b, wqkv, lwo, l2g, l2b, lw1, lb1, lw2, lb2) = \
            w[8 + 10 * li:18 + 10 * li]
        hh = _ln(lat, l1g[...], l1b[...])
        qkv = _mm(hh, wqkv[...])
        merged = []
        for bb in range(_B):
            qb_ = qkv[bb * _L:(bb + 1) * _L, :_D]
            kb_ = qkv[bb * _L:(bb + 1) * _L, _D:2 * _D]
            vb_ = qkv[bb * _L:(bb + 1) * _L, 2 * _D:]
            p = _softmax_rows(_mm_t(_block_diag(qb_, _L), kb_) / _SQRTDH)
            merged.append(_unblock(_mm(p, vb_), _L))
        lat = lat + _mm(jnp.concatenate(merged, axis=0), lwo[...])
        hh = _ln(lat, l2g[...], l2b[...])
        lat = lat + _mm(jax.nn.gelu(_mm(hh, lw1[...]) + lb1[...]), lw2[...]) + lb2[...]

    (bquery, bwq, bwkv, bwoT, blng, blnb) = w[28:34]
    nl2 = _ln(lat, blng[...], blnb[...])
    kvb = _mm(nl2, bwkv[...])
    qb = _mm(bquery[...], bwq[...])
    qbig = _block_diag(qb, _BD)
    bwo_row = bwoT[...].astype(jnp.bfloat16).astype(jnp.float32)
    rows = []
    for bb in range(_B):
        kb_ = kvb[bb * _L:(bb + 1) * _L, :_D]
        vb_ = kvb[bb * _L:(bb + 1) * _L, _D:]
        p = _softmax_rows(_mm_t(qbig, kb_) / _SQRTDH)
        o = _mm(p, vb_)
        attnb = jnp.concatenate(
            [o[h * _BD:(h + 1) * _BD, h * _DH:(h + 1) * _DH] for h in range(_H)],
            axis=1)
        prod = attnb.astype(jnp.bfloat16).astype(jnp.float32) * bwo_row
        vals = [jnp.sum(prod[d:d + 1, :], axis=1, keepdims=True)
                for d in range(_BD)]
        rows.append(jnp.concatenate(vals, axis=1))
    out_ref[...] = jnp.concatenate(rows, axis=0)


def _tc_tail(attn_all, weights):
    return pl.pallas_call(
        _tail_body,
        out_shape=jax.ShapeDtypeStruct((_B, _BD), jnp.float32),
    )(attn_all, *weights)


def kernel(params, neuron_ids, time_bins, values, batch_indices):
    p = params
    flat = _sc_gather(neuron_ids, time_bins, values,
                      p['neuron_emb'], p['time_emb'], p['value_emb'])
    r = lambda a: a.reshape(1, -1)
    c = p['cross']
    bh = p['beh']
    cross_w = [p['latents'], r(p['tok_ln_g']), r(p['tok_ln_b']),
               r(c['lnq_g']), r(c['lnq_b']), r(c['lnk_g']), r(c['lnk_b']),
               c['wq'], c['wkv'][:, :_D], c['wkv'][:, _D:]]
    tail_w = [p['latents'], c['wo'], r(c['ln2_g']), r(c['ln2_b']),
              c['w1'], r(c['b1']), c['w2'], r(c['b2'])]
    for lyr in p['layers']:
        tail_w += [r(lyr['ln1_g']), r(lyr['ln1_b']), lyr['wqkv'], lyr['wo'],
                   r(lyr['ln2_g']), r(lyr['ln2_b']), lyr['w1'], r(lyr['b1']),
                   lyr['w2'], r(lyr['b2'])]
    tail_w += [bh['query'], bh['wq'], bh['wkv'], bh['wo'].reshape(1, _D),
               r(bh['ln_g']), r(bh['ln_b'])]
    assert len(cross_w) == _NWC and len(tail_w) == _NWT
    bi3d = batch_indices.astype(jnp.int32).reshape(_NCHUNK, 1, _CHUNK)
    attn_all = _tc_cross(bi3d, flat, cross_w)
    return _tc_tail(attn_all, tail_w)
